# Initial kernel scaffold; baseline (speedup 1.0000x reference)
import jax
import jax.numpy as jnp
from jax.experimental import pallas as pl


def kernel(x, edge_index, edge_type, weight, comp, root, bias, weight_mask):
    raise NotImplementedError("write your pallas kernel here")


# speedup vs baseline: 1.8013x; 1.8013x over previous
import jax
import jax.numpy as jnp
from jax import lax
from jax.experimental import pallas as pl
from jax.experimental.pallas import tpu as pltpu
from jax.experimental.pallas import tpu_sc as plsc

N = 10000
NPAD = 10240
C = 128
E = 320000
R = 8
NBASES = 18
NS = 16
NC = 2
NH = 2
HROWS = NPAD // NH
ACCR = HROWS + 128
DUMPL = HROWS
EPT = E // NS
G = 80
NGRP = 5
CH = G * NGRP
NCHUNK = EPT // CH
RPT = ACCR // NS
NBLK = 10
BLK = NPAD // NBLK


def _sc_body(x_hbm, src_hbm, dst_hbm, et_hbm, acc_hbm, cnt_hbm,
             meta_src, meta_dst, meta_et,
             gsrc0, gsrc1, gsrc2, gsrc3, gsrc4,
             gidx0, gidx1, gidx2, gidx3, gidx4,
             rows0, rows1, rows2, rows3, rows4,
             ones_buf, zrow, zcnt, acc_sh, cnt_sh,
             sem0, sem1, sem2, sem3, sem4):
    core = lax.axis_index("c")
    tid = lax.axis_index("s")
    sems = [sem0, sem1, sem2, sem3, sem4]
    gsrc = [gsrc0, gsrc1, gsrc2, gsrc3, gsrc4]
    gidx = [gidx0, gidx1, gidx2, gidx3, gidx4]
    rows = [rows0, rows1, rows2, rows3, rows4]
    zeros16 = jnp.zeros((16,), jnp.float32)
    ones16 = jnp.ones((16,), jnp.float32)
    dump16 = jnp.full((16,), DUMPL, jnp.int32)

    def _fill_z(i, _):
        for j in range(8):
            zrow[i, pl.ds(j * 16, 16)] = zeros16
        zcnt[i, :] = zeros16
        return 0
    lax.fori_loop(0, 64, _fill_z, 0)

    def _fill_o(i, _):
        ones_buf[i, :] = ones16
        return 0
    lax.fori_loop(0, G, _fill_o, 0)

    row0 = tid * RPT
    ebase = tid * EPT

    def _round(i, _):
        r = core * 4 + i // NH
        h = i % NH
        node0 = h * HROWS

        for k in range(5):
            pltpu.sync_copy(zrow, acc_sh.at[pl.ds(row0 + k * 64, 64)])
            pltpu.sync_copy(zcnt, cnt_sh.at[pl.ds(row0 + k * 64, 64)])
        pltpu.sync_copy(zrow.at[pl.ds(0, RPT - 320)],
                        acc_sh.at[pl.ds(row0 + 320, RPT - 320)])
        pltpu.sync_copy(zcnt.at[pl.ds(0, RPT - 320)],
                        cnt_sh.at[pl.ds(row0 + 320, RPT - 320)])
        plsc.subcore_barrier()

        def _chunk(c, _):
            base = ebase + c * CH
            pltpu.sync_copy(src_hbm.at[pl.ds(base, CH)], meta_src)
            pltpu.sync_copy(dst_hbm.at[pl.ds(base, CH)], meta_dst)
            pltpu.sync_copy(et_hbm.at[pl.ds(base, CH)], meta_et)
            cps = []
            for g in range(NGRP):
                for v in range(G // 16):
                    off = g * G + v * 16
                    s16 = meta_src[pl.ds(off, 16)]
                    d16 = meta_dst[pl.ds(off, 16)]
                    t16 = meta_et[pl.ds(off, 16)]
                    dl = d16 - node0
                    m = (t16 == r) & (dl >= 0) & (dl < HROWS)
                    gsrc[g][pl.ds(v * 16, 16)] = s16
                    gidx[g][pl.ds(v * 16, 16)] = jnp.where(m, dl, dump16)
                cps.append(
                    pltpu.async_copy(x_hbm.at[gsrc[g]], rows[g], sems[g]))
            for g in range(NGRP):
                cps[g].wait()
                pltpu.sync_copy(rows[g], acc_sh.at[gidx[g]], add=True)
                pltpu.sync_copy(ones_buf, cnt_sh.at[gidx[g]], add=True)
            return 0
        lax.fori_loop(0, NCHUNK, _chunk, 0)

        plsc.subcore_barrier()
        pltpu.sync_copy(acc_sh.at[pl.ds(row0, RPT)],
                        acc_hbm.at[r, h, pl.ds(row0, RPT)])
        pltpu.sync_copy(cnt_sh.at[pl.ds(row0, RPT)],
                        cnt_hbm.at[r, h, pl.ds(row0, RPT)])
        plsc.subcore_barrier()
        return 0

    lax.fori_loop(0, 4 * NH, _round, 0)


_sc_agg = pl.kernel(
    _sc_body,
    out_type=(jax.ShapeDtypeStruct((R, NH, ACCR, C), jnp.float32),
              jax.ShapeDtypeStruct((R, NH, ACCR, 16), jnp.float32)),
    mesh=plsc.VectorSubcoreMesh(core_axis_name="c", subcore_axis_name="s",
                                num_cores=NC, num_subcores=NS),
    compiler_params=pltpu.CompilerParams(use_tc_tiling_on_sc=False),
    scratch_types=[
        pltpu.VMEM((CH,), jnp.int32),
        pltpu.VMEM((CH,), jnp.int32),
        pltpu.VMEM((CH,), jnp.int32),
        pltpu.VMEM((G,), jnp.int32),
        pltpu.VMEM((G,), jnp.int32),
        pltpu.VMEM((G,), jnp.int32),
        pltpu.VMEM((G,), jnp.int32),
        pltpu.VMEM((G,), jnp.int32),
        pltpu.VMEM((G,), jnp.int32),
        pltpu.VMEM((G,), jnp.int32),
        pltpu.VMEM((G,), jnp.int32),
        pltpu.VMEM((G,), jnp.int32),
        pltpu.VMEM((G,), jnp.int32),
        pltpu.VMEM((G, C), jnp.float32),
        pltpu.VMEM((G, C), jnp.float32),
        pltpu.VMEM((G, C), jnp.float32),
        pltpu.VMEM((G, C), jnp.float32),
        pltpu.VMEM((G, C), jnp.float32),
        pltpu.VMEM((G, 16), jnp.float32),
        pltpu.VMEM((64, C), jnp.float32),
        pltpu.VMEM((64, 16), jnp.float32),
        pltpu.VMEM_SHARED((ACCR, C), jnp.float32),
        pltpu.VMEM_SHARED((ACCR, 16), jnp.float32),
        pltpu.SemaphoreType.DMA,
        pltpu.SemaphoreType.DMA,
        pltpu.SemaphoreType.DMA,
        pltpu.SemaphoreType.DMA,
        pltpu.SemaphoreType.DMA,
    ],
)


def _tc_body(cm_ref, bias_ref, acc_ref, cnt_ref, x_ref, weight_ref, root_ref,
             out_ref, w_s):
    i = pl.program_id(0)

    @pl.when(i == 0)
    def _():
        for r in range(R):
            wacc = jnp.zeros((C, C), jnp.float32)
            for b in range(NBASES):
                wacc = wacc + cm_ref[r, b] * weight_ref[b]
            w_s[r] = wacc

    o = jnp.dot(x_ref[:], root_ref[:],
                preferred_element_type=jnp.float32) + bias_ref[0:1, :]
    for r in range(R):
        inv = 1.0 / jnp.maximum(cnt_ref[r, 0, :, 0:1], 1.0)
        h = acc_ref[r, 0] * inv
        o = o + jnp.dot(h, w_s[r], preferred_element_type=jnp.float32)
    out_ref[:] = o


def _tc_finish(cm, bias2, acc, cnt, xpad, weight, root):
    nb = HROWS // BLK
    return pl.pallas_call(
        _tc_body,
        grid=(NBLK,),
        in_specs=[
            pl.BlockSpec(memory_space=pltpu.SMEM),
            pl.BlockSpec((R, C), lambda i: (0, 0)),
            pl.BlockSpec((R, 1, BLK, C),
                         lambda i: (0, i // nb, i % nb, 0)),
            pl.BlockSpec((R, 1, BLK, 16),
                         lambda i: (0, i // nb, i % nb, 0)),
            pl.BlockSpec((BLK, C), lambda i: (i, 0)),
            pl.BlockSpec((NBASES, C, C), lambda i: (0, 0, 0)),
            pl.BlockSpec((C, C), lambda i: (0, 0)),
        ],
        out_specs=pl.BlockSpec((BLK, C), lambda i: (i, 0)),
        out_shape=jax.ShapeDtypeStruct((NPAD, C), jnp.float32),
        scratch_shapes=[pltpu.VMEM((R, C, C), jnp.float32)],
    )(cm, bias2, acc, cnt, xpad, weight, root)


def kernel(x, edge_index, edge_type, weight, comp, root, bias, weight_mask):
    src = edge_index[0]
    dst = edge_index[1]
    cm = comp * weight_mask
    acc, cnt = _sc_agg(x, src, dst, edge_type)
    xpad = jnp.pad(x, ((0, NPAD - N), (0, 0)))
    bias2 = jnp.broadcast_to(bias, (R, C))
    out = _tc_finish(cm, bias2, acc, cnt, xpad, weight, root)
    return out[:N]
